# baseline (speedup 1.0000x reference)
import functools

import jax
import jax.numpy as jnp
from jax import lax
from jax.experimental import pallas as pl
from jax.experimental.pallas import tpu as pltpu
from jax.experimental.pallas import tpu_sc as plsc

B, L, D = 4096, 200, 64
N = B * L
NC, NS = 2, 16
NW = NC * NS
STEP = 128
TPW = N // NW
NSTEPS = TPW // STEP
NROWS = N // STEP
NBUF = 4


def _emb_body(tok_hbm, w0_hbm, w1_hbm, w2_hbm,
              out_hbm, tok_v, idx_v, rows, sem_idx, sem_in, sem_out):
    cid = lax.axis_index("c")
    sid = lax.axis_index("s")
    wid = sid * NC + cid
    obase = wid * TPW
    whbm = (w0_hbm, w1_hbm, w2_hbm)

    def fire_idx(t, b):
        pltpu.async_copy(tok_hbm.at[pl.ds(obase + t * STEP, STEP)],
                         tok_v.at[b], sem_idx.at[b])

    def drain_idx(b):
        pltpu.make_async_copy(tok_hbm.at[pl.ds(obase, STEP)],
                              tok_v.at[b], sem_idx.at[b]).wait()

    def extract_idx(b):
        iota = lax.iota(jnp.int32, 16)
        for g in range(STEP // 16):
            rid = iota + (16 * g)
            for f in range(3):
                col = jnp.full((16,), f, jnp.int32)
                idx_v[b, f, pl.ds(16 * g, 16)] = plsc.load_gather(
                    tok_v.at[b], [rid, col])

    def fire_gather(b):
        for f in range(3):
            pltpu.async_copy(whbm[f].at[idx_v.at[b, f]], rows.at[b, f],
                             sem_in.at[b])

    def drain_gather(b):
        for f in range(3):
            pltpu.make_async_copy(whbm[f].at[idx_v.at[b, f]], rows.at[b, f],
                                  sem_in.at[b]).wait()

    def accum(b):
        @plsc.parallel_loop(0, STEP, unroll=4)
        def addrow(i):
            for k in range(D // 16):
                sl = pl.ds(k * 16, 16)
                plsc.addupdate(rows.at[b, 0, i, sl],
                               rows[b, 1, i, sl] + rows[b, 2, i, sl])

    def fire_out(t, b):
        pltpu.async_copy(rows.at[b, 0],
                         out_hbm.at[pl.ds(obase + t * STEP, STEP)],
                         sem_out.at[b])

    def drain_out(b):
        pltpu.make_async_copy(rows.at[b, 0],
                              out_hbm.at[pl.ds(obase, STEP)],
                              sem_out.at[b]).wait()

    for t in range(NBUF):
        fire_idx(t, t)
    for t in range(2):
        drain_idx(t)
        extract_idx(t)
        fire_gather(t)

    def body4(u, carry):
        t0 = NBUF * u
        for b in range(NBUF):
            t = t0 + b

            @pl.when(t < NSTEPS - 2)
            def _():
                @pl.when(t >= 2)
                def _():
                    drain_out((b + 2) % NBUF)
                drain_idx((b + 2) % NBUF)
                extract_idx((b + 2) % NBUF)
                fire_gather((b + 2) % NBUF)

            drain_gather(b)

            @pl.when(t < NSTEPS - NBUF)
            def _():
                fire_idx(t + NBUF, b)

            accum(b)
            fire_out(t, b)
        return carry

    lax.fori_loop(0, NSTEPS // NBUF, body4, 0)
    for b in range(NBUF):
        drain_out(b)


@functools.partial(jax.jit, static_argnames=())
def _emb_call(tok, W0, W1, W2):
    mesh = plsc.VectorSubcoreMesh(core_axis_name="c", subcore_axis_name="s")
    return pl.kernel(
        _emb_body,
        out_type=jax.ShapeDtypeStruct((N, D), jnp.float32),
        mesh=mesh,
        scratch_types=[
            pltpu.VMEM((NBUF, STEP, 3), jnp.int32),
            pltpu.VMEM((NBUF, 3, STEP), jnp.int32),
            pltpu.VMEM((NBUF, 3, STEP, D), jnp.float32),
            pltpu.SemaphoreType.DMA((NBUF,)),
            pltpu.SemaphoreType.DMA((NBUF,)),
            pltpu.SemaphoreType.DMA((NBUF,)),
        ],
        compiler_params=pltpu.CompilerParams(use_tc_tiling_on_sc=False,
                                             needs_layout_passes=False),
    )(tok, W0, W1, W2)


def kernel(factored_tokens, W0, W1, W2):
    tok = factored_tokens.reshape(N, 3).astype(jnp.int32)
    out = _emb_call(tok, W0, W1, W2)
    return out.reshape(B, L, D)


# speedup vs baseline: 1.0725x; 1.0725x over previous
import functools

import jax
import jax.numpy as jnp
from jax import lax
from jax.experimental import pallas as pl
from jax.experimental.pallas import tpu as pltpu
from jax.experimental.pallas import tpu_sc as plsc

B, L, D = 4096, 200, 64
N = B * L
NC, NS = 2, 16
NW = NC * NS
STEP = 128
TPW = N // NW
NSTEPS = TPW // STEP
NROWS = N // STEP
NBUF = 4


def _emb_body(tok_hbm, w0_hbm, w1_hbm, w2_hbm,
              out_hbm, tok_v, idx_v, rows, sem_idx, sem_in, sem_out):
    cid = lax.axis_index("c")
    sid = lax.axis_index("s")
    wid = sid * NC + cid
    obase = wid * TPW
    whbm = (w0_hbm, w1_hbm, w2_hbm)

    def fire_idx(t, b):
        pltpu.async_copy(tok_hbm.at[pl.ds((obase + t * STEP) * 3, STEP * 3)],
                         tok_v.at[b], sem_idx.at[b])

    def drain_idx(b):
        pltpu.make_async_copy(tok_hbm.at[pl.ds(obase * 3, STEP * 3)],
                              tok_v.at[b], sem_idx.at[b]).wait()

    def extract_idx(b):
        iota3 = lax.iota(jnp.int32, 16) * 3
        for g in range(STEP // 16):
            for f in range(3):
                idx_v[b, f, pl.ds(16 * g, 16)] = plsc.load_gather(
                    tok_v.at[b], [iota3 + (48 * g + f)])

    def fire_gather(b):
        for f in range(3):
            pltpu.async_copy(whbm[f].at[idx_v.at[b, f]], rows.at[b, f],
                             sem_in.at[b])

    def drain_gather(b):
        for f in range(3):
            pltpu.make_async_copy(whbm[f].at[idx_v.at[b, f]], rows.at[b, f],
                                  sem_in.at[b]).wait()

    def accum(b):
        @plsc.parallel_loop(0, STEP, unroll=4)
        def addrow(i):
            for k in range(D // 16):
                sl = pl.ds(k * 16, 16)
                plsc.addupdate(rows.at[b, 0, i, sl],
                               rows[b, 1, i, sl] + rows[b, 2, i, sl])

    def fire_out(t, b):
        pltpu.async_copy(rows.at[b, 0],
                         out_hbm.at[pl.ds(obase + t * STEP, STEP)],
                         sem_out.at[b])

    def drain_out(b):
        pltpu.make_async_copy(rows.at[b, 0],
                              out_hbm.at[pl.ds(obase, STEP)],
                              sem_out.at[b]).wait()

    for t in range(NBUF):
        fire_idx(t, t)
    for t in range(2):
        drain_idx(t)
        extract_idx(t)
        fire_gather(t)

    def body4(u, carry):
        t0 = NBUF * u
        for b in range(NBUF):
            t = t0 + b

            @pl.when(t < NSTEPS - 2)
            def _():
                @pl.when(t >= 2)
                def _():
                    drain_out((b + 2) % NBUF)
                drain_idx((b + 2) % NBUF)
                extract_idx((b + 2) % NBUF)
                fire_gather((b + 2) % NBUF)

            drain_gather(b)

            @pl.when(t < NSTEPS - NBUF)
            def _():
                fire_idx(t + NBUF, b)

            accum(b)
            fire_out(t, b)
        return carry

    lax.fori_loop(0, NSTEPS // NBUF, body4, 0)
    for b in range(NBUF):
        drain_out(b)


@functools.partial(jax.jit, static_argnames=())
def _emb_call(tok, W0, W1, W2):
    mesh = plsc.VectorSubcoreMesh(core_axis_name="c", subcore_axis_name="s")
    return pl.kernel(
        _emb_body,
        out_type=jax.ShapeDtypeStruct((N, D), jnp.float32),
        mesh=mesh,
        scratch_types=[
            pltpu.VMEM((NBUF, STEP * 3), jnp.int32),
            pltpu.VMEM((NBUF, 3, STEP), jnp.int32),
            pltpu.VMEM((NBUF, 3, STEP, D), jnp.float32),
            pltpu.SemaphoreType.DMA((NBUF,)),
            pltpu.SemaphoreType.DMA((NBUF,)),
            pltpu.SemaphoreType.DMA((NBUF,)),
        ],
        compiler_params=pltpu.CompilerParams(use_tc_tiling_on_sc=False,
                                             needs_layout_passes=False),
    )(tok, W0, W1, W2)


def kernel(factored_tokens, W0, W1, W2):
    tok = factored_tokens.reshape(N * 3).astype(jnp.int32)
    out = _emb_call(tok, W0, W1, W2)
    return out.reshape(B, L, D)


# speedup vs baseline: 3.7769x; 3.5217x over previous
import functools

import jax
import jax.numpy as jnp
from jax import lax
from jax.experimental import pallas as pl
from jax.experimental.pallas import tpu as pltpu
from jax.experimental.pallas import tpu_sc as plsc

B, L, D = 4096, 200, 64
N = B * L
NC, NS = 2, 16
NW = NC * NS
STEP = 128
TPW = N // NW
NSTEPS = TPW // STEP
NROWS = N // STEP
NBUF = 4


def _emb_body(idx0_hbm, idx1_hbm, idx2_hbm, w0_hbm, w1_hbm, w2_hbm,
              out_hbm, idx_v, rows, sem_idx, sem_in, sem_out):
    cid = lax.axis_index("c")
    sid = lax.axis_index("s")
    wid = sid * NC + cid
    rbase = wid * NSTEPS
    obase = wid * TPW
    whbm = (w0_hbm, w1_hbm, w2_hbm)
    ihbm = (idx0_hbm, idx1_hbm, idx2_hbm)

    def fire_idx(t, b):
        for f in range(3):
            pltpu.async_copy(ihbm[f].at[rbase + t], idx_v.at[b, f],
                             sem_idx.at[b])

    def drain_idx(b):
        for f in range(3):
            pltpu.make_async_copy(ihbm[f].at[rbase], idx_v.at[b, f],
                                  sem_idx.at[b]).wait()

    def fire_gather(b):
        for f in range(3):
            pltpu.async_copy(whbm[f].at[idx_v.at[b, f]], rows.at[b, f],
                             sem_in.at[b])

    def drain_gather(b):
        for f in range(3):
            pltpu.make_async_copy(whbm[f].at[idx_v.at[b, f]], rows.at[b, f],
                                  sem_in.at[b]).wait()

    def accum(b):
        @plsc.parallel_loop(0, STEP, unroll=4)
        def addrow(i):
            for k in range(D // 16):
                sl = pl.ds(k * 16, 16)
                plsc.addupdate(rows.at[b, 0, i, sl],
                               rows[b, 1, i, sl] + rows[b, 2, i, sl])

    def fire_out(t, b):
        pltpu.async_copy(rows.at[b, 0],
                         out_hbm.at[pl.ds(obase + t * STEP, STEP)],
                         sem_out.at[b])

    def drain_out(b):
        pltpu.make_async_copy(rows.at[b, 0],
                              out_hbm.at[pl.ds(obase, STEP)],
                              sem_out.at[b]).wait()

    for t in range(NBUF):
        fire_idx(t, t)
    for t in range(2):
        drain_idx(t)
        fire_gather(t)

    def body4(u, carry):
        t0 = NBUF * u
        for b in range(NBUF):
            t = t0 + b

            @pl.when(t < NSTEPS - 2)
            def _():
                @pl.when(t >= 2)
                def _():
                    drain_out((b + 2) % NBUF)
                drain_idx((b + 2) % NBUF)
                fire_gather((b + 2) % NBUF)

            drain_gather(b)

            @pl.when(t < NSTEPS - NBUF)
            def _():
                fire_idx(t + NBUF, b)

            accum(b)
            fire_out(t, b)
        return carry

    lax.fori_loop(0, NSTEPS // NBUF, body4, 0)
    for b in range(NBUF):
        drain_out(b)


@functools.partial(jax.jit, static_argnames=())
def _emb_call(idx0, idx1, idx2, W0, W1, W2):
    mesh = plsc.VectorSubcoreMesh(core_axis_name="c", subcore_axis_name="s")
    return pl.kernel(
        _emb_body,
        out_type=jax.ShapeDtypeStruct((N, D), jnp.float32),
        mesh=mesh,
        scratch_types=[
            pltpu.VMEM((NBUF, 3, STEP), jnp.int32),
            pltpu.VMEM((NBUF, 3, STEP, D), jnp.float32),
            pltpu.SemaphoreType.DMA((NBUF,)),
            pltpu.SemaphoreType.DMA((NBUF,)),
            pltpu.SemaphoreType.DMA((NBUF,)),
        ],
        compiler_params=pltpu.CompilerParams(use_tc_tiling_on_sc=False),
    )(idx0, idx1, idx2, W0, W1, W2)


def kernel(factored_tokens, W0, W1, W2):
    ft = factored_tokens.reshape(N, 3).astype(jnp.int32)
    idx = ft.T.reshape(3, NROWS, STEP)
    out = _emb_call(idx[0], idx[1], idx[2], W0, W1, W2)
    return out.reshape(B, L, D)
